# Initial kernel scaffold; baseline (speedup 1.0000x reference)
import jax
import jax.numpy as jnp
from jax.experimental import pallas as pl


def kernel(x, edge_index, W_l, b_l, W_r):
    raise NotImplementedError("write your pallas kernel here")


# speedup vs baseline: 2.2252x; 2.2252x over previous
import functools

import jax
import jax.numpy as jnp
from jax import lax
from jax.experimental import pallas as pl
from jax.experimental.pallas import tpu as pltpu
from jax.experimental.pallas import tpu_sc as plsc

N_NODES = 10000
N_EDGES = 320000
D_IN = 128
D_OUT = 128

NUM_CORES = 2
NUM_SUBCORES = 16

CHUNK = 64
CHUNKS_PER_TILE = 320
EDGES_PER_TILE = CHUNK * CHUNKS_PER_TILE
E_PAD = EDGES_PER_TILE * NUM_SUBCORES
N_PAD = 10240
HALF_N = N_PAD // NUM_CORES
GARBAGE = 128
ACC_ROWS = HALF_N + GARBAGE
ZROWS = ACC_ROWS // NUM_SUBCORES
WROWS = HALF_N // NUM_SUBCORES
SROWS = ZROWS // 4


def _sc_aggregate(x, srcdst, ones_blk, zeros_blk):
  mesh = plsc.VectorSubcoreMesh(core_axis_name="c", subcore_axis_name="s")

  @functools.partial(
      pl.kernel,
      out_type=(
          jax.ShapeDtypeStruct((N_PAD, D_IN), jnp.float32),
          jax.ShapeDtypeStruct((N_PAD, D_IN), jnp.float32),
      ),
      mesh=mesh,
      scratch_types=[
          pltpu.VMEM((2, CHUNK), jnp.int32),
          pltpu.VMEM((2, CHUNK), jnp.int32),
          pltpu.VMEM((2, CHUNK), jnp.int32),
          pltpu.VMEM((2, CHUNK, D_IN), jnp.float32),
          pltpu.VMEM((CHUNK, D_IN), jnp.float32),
          pltpu.VMEM((SROWS, D_IN), jnp.float32),
          pltpu.VMEM_SHARED((ACC_ROWS, D_IN), jnp.float32),
          pltpu.VMEM_SHARED((ACC_ROWS, D_IN), jnp.float32),
          pltpu.SemaphoreType.DMA,
          pltpu.SemaphoreType.DMA,
      ],
  )
  def k(x_hbm, sd_hbm, ones_hbm, z_hbm, psum_hbm, pdeg_hbm,
        src_v, dst_v, dloc_v, rows_v, ones_v, stage_v, acc_sh, accd_sh,
        sem0, sem1):
    c = lax.axis_index("c")
    s = lax.axis_index("s")
    sems = (sem0, sem1)

    zbase = s * ZROWS
    pltpu.sync_copy(ones_hbm, ones_v)
    pltpu.sync_copy(z_hbm, stage_v)
    for kk in range(4):
      pltpu.sync_copy(stage_v, acc_sh.at[pl.ds(zbase + kk * SROWS, SROWS)])
      pltpu.sync_copy(stage_v, accd_sh.at[pl.ds(zbase + kk * SROWS, SROWS)])
    plsc.subcore_barrier()

    is0 = c == 0

    def load_and_issue(j, b):
      pltpu.sync_copy(sd_hbm.at[0, s, j], src_v.at[b])
      pltpu.sync_copy(sd_hbm.at[1, s, j], dst_v.at[b])
      for kk in range(CHUNK // 16):
        d = dst_v[b, pl.ds(kk * 16, 16)]
        lo = jnp.minimum(d, HALF_N)
        hi = jnp.maximum(d - (HALF_N - GARBAGE), 0)
        dloc_v[b, pl.ds(kk * 16, 16)] = jnp.where(is0, lo, hi)
      return pltpu.async_copy(x_hbm.at[src_v.at[b]], rows_v.at[b], sems[b])

    def drain(b):
      pltpu.sync_copy(rows_v.at[b], acc_sh.at[dloc_v.at[b]], add=True)
      pltpu.sync_copy(ones_v, accd_sh.at[dloc_v.at[b]], add=True)

    cp0 = load_and_issue(0, 0)
    cp1 = load_and_issue(1, 1)

    def step(kstep, carry):
      for b in range(2):
        j = 2 * kstep + b
        pltpu.make_async_copy(
            x_hbm.at[src_v.at[b]], rows_v.at[b], sems[b]).wait()
        drain(b)
        load_and_issue(j + 2, b)
      return carry

    lax.fori_loop(0, CHUNKS_PER_TILE // 2 - 1, step, 0)
    for b in range(2):
      pltpu.make_async_copy(
          x_hbm.at[src_v.at[b]], rows_v.at[b], sems[b]).wait()
      drain(b)
    plsc.subcore_barrier()

    abase = c * GARBAGE + s * WROWS
    obase = c * HALF_N + s * WROWS
    ww = WROWS // 4
    for kk in range(4):
      pltpu.sync_copy(acc_sh.at[pl.ds(abase + kk * ww, ww)],
                      stage_v.at[pl.ds(0, ww)])
      pltpu.sync_copy(stage_v.at[pl.ds(0, ww)],
                      psum_hbm.at[pl.ds(obase + kk * ww, ww)])
      pltpu.sync_copy(accd_sh.at[pl.ds(abase + kk * ww, ww)],
                      stage_v.at[pl.ds(0, ww)])
      pltpu.sync_copy(stage_v.at[pl.ds(0, ww)],
                      pdeg_hbm.at[pl.ds(obase + kk * ww, ww)])

  return k(x, srcdst, ones_blk, zeros_blk)


BLK_N = 1000


def _tc_body(p_ref, d_ref, x_ref, wl_ref, wr_ref, b_ref, o_ref):
  deg = jnp.maximum(d_ref[...][:, 0:1], 1.0)
  aggr = p_ref[...] / deg
  o_ref[...] = (
      jnp.dot(aggr, wl_ref[...], preferred_element_type=jnp.float32)
      + jnp.dot(x_ref[...], wr_ref[...], preferred_element_type=jnp.float32)
      + b_ref[...]
  )


def kernel(x, edge_index, W_l, b_l, W_r):
  pad = E_PAD - N_EDGES
  src = jnp.concatenate([edge_index[0], jnp.zeros((pad,), jnp.int32)])
  dst = jnp.concatenate(
      [edge_index[1], jnp.full((pad,), N_NODES, jnp.int32)])
  srcdst = jnp.stack([src, dst]).reshape(
      2, NUM_SUBCORES, CHUNKS_PER_TILE, CHUNK)

  ones_blk = jnp.ones((CHUNK, D_IN), jnp.float32)
  zeros_blk = jnp.zeros((SROWS, D_IN), jnp.float32)

  psum, pdeg = _sc_aggregate(x, srcdst, ones_blk, zeros_blk)

  grid = N_NODES // BLK_N
  out = pl.pallas_call(
      _tc_body,
      grid=(grid,),
      in_specs=[
          pl.BlockSpec((BLK_N, D_IN), lambda i: (i, 0)),
          pl.BlockSpec((BLK_N, D_IN), lambda i: (i, 0)),
          pl.BlockSpec((BLK_N, D_IN), lambda i: (i, 0)),
          pl.BlockSpec((D_IN, D_OUT), lambda i: (0, 0)),
          pl.BlockSpec((D_IN, D_OUT), lambda i: (0, 0)),
          pl.BlockSpec((1, D_OUT), lambda i: (0, 0)),
      ],
      out_specs=pl.BlockSpec((BLK_N, D_OUT), lambda i: (i, 0)),
      out_shape=jax.ShapeDtypeStruct((N_NODES, D_OUT), jnp.float32),
  )(psum, pdeg, x, W_l.T, W_r.T, b_l.reshape(1, D_OUT))
  return out


# speedup vs baseline: 4.2629x; 1.9158x over previous
import functools

import jax
import jax.numpy as jnp
from jax import lax
from jax.experimental import pallas as pl
from jax.experimental.pallas import tpu as pltpu
from jax.experimental.pallas import tpu_sc as plsc

N_NODES = 10000
N_EDGES = 320000
D_IN = 128
D_OUT = 128

NUM_CORES = 2
NUM_SUBCORES = 16

CHUNK = 64
NB = 8
CHUNKS_PER_TILE = 320
NBLOCKS = CHUNKS_PER_TILE // NB
EDGES_PER_TILE = CHUNK * CHUNKS_PER_TILE
E_PAD = EDGES_PER_TILE * NUM_SUBCORES
N_PAD = 10240
ZROWS = N_PAD // NUM_SUBCORES
SROWS = ZROWS // 8


def _sc_aggregate(x, srcdst, ones_blk, zeros_blk):
  mesh = plsc.VectorSubcoreMesh(core_axis_name="c", subcore_axis_name="s")

  @functools.partial(
      pl.kernel,
      out_type=(
          jax.ShapeDtypeStruct((N_PAD, D_IN), jnp.float32),
          jax.ShapeDtypeStruct((N_PAD, D_IN), jnp.float32),
      ),
      mesh=mesh,
      scratch_types=[
          pltpu.VMEM((2, NB, CHUNK), jnp.int32),
          pltpu.VMEM((2, NB, CHUNK), jnp.int32),
          pltpu.VMEM((2, CHUNK, D_IN), jnp.float32),
          pltpu.VMEM((CHUNK, D_IN), jnp.float32),
          pltpu.VMEM((SROWS, D_IN), jnp.float32),
          pltpu.VMEM_SHARED((N_PAD, D_IN), jnp.float32),
          pltpu.SemaphoreType.DMA,
          pltpu.SemaphoreType.DMA,
          pltpu.SemaphoreType.DMA,
          pltpu.SemaphoreType.DMA,
      ],
  )
  def k(x_hbm, sd_hbm, ones_hbm, z_hbm, psum_hbm, pdeg_hbm,
        srcb_v, dstb_v, rows_v, ones_v, stage_v, acc_sh,
        rsem0, rsem1, isem0, isem1):
    c = lax.axis_index("c")
    s = lax.axis_index("s")
    rsems = (rsem0, rsem1)
    isems = (isem0, isem1)

    zbase = s * ZROWS
    pltpu.sync_copy(ones_hbm, ones_v)
    pltpu.sync_copy(z_hbm, stage_v)
    for kk in range(8):
      pltpu.sync_copy(stage_v, acc_sh.at[pl.ds(zbase + kk * SROWS, SROWS)])
    plsc.subcore_barrier()

    def load_block(g, slot):
      pltpu.async_copy(sd_hbm.at[0, s, g], srcb_v.at[slot], isems[slot])
      pltpu.async_copy(sd_hbm.at[1, s, g], dstb_v.at[slot], isems[slot])

    def wait_block(slot):
      pltpu.make_async_copy(sd_hbm.at[0, s, 0], srcb_v.at[slot],
                            isems[slot]).wait()
      pltpu.make_async_copy(sd_hbm.at[1, s, 0], dstb_v.at[slot],
                            isems[slot]).wait()

    @pl.when(c == 0)
    def _sum_pipeline():
      def fire(bslot, jj, rslot):
        pltpu.async_copy(x_hbm.at[srcb_v.at[bslot, jj]], rows_v.at[rslot],
                         rsems[rslot])

      def wait_rows(rslot):
        pltpu.make_async_copy(x_hbm.at[srcb_v.at[0, 0]], rows_v.at[rslot],
                              rsems[rslot]).wait()

      def drain(bslot, jj, rslot):
        pltpu.sync_copy(rows_v.at[rslot], acc_sh.at[dstb_v.at[bslot, jj]],
                        add=True)

      load_block(0, 0)
      load_block(1, 1)
      wait_block(0)
      fire(0, 0, 0)
      fire(0, 1, 1)

      def step(p, carry):
        for jj in range(16):
          rslot = jj & 1
          bslot, bjj = (0, jj) if jj < 8 else (1, jj - 8)
          wait_rows(rslot)
          drain(bslot, bjj, rslot)
          nxt = jj + 2
          if nxt < 8:
            fire(0, nxt, rslot)
          elif nxt == 8:
            wait_block(1)
            fire(1, 0, rslot)
          elif nxt == 9:
            load_block(2 * p + 2, 0)
            fire(1, 1, rslot)
          elif nxt < 16:
            fire(1, nxt - 8, rslot)
          elif nxt == 16:
            wait_block(0)
            fire(0, 0, rslot)
          else:
            load_block(2 * p + 3, 1)
            fire(0, 1, rslot)
        return carry

      lax.fori_loop(0, NBLOCKS // 2 - 1, step, 0)

      for jj in range(16):
        rslot = jj & 1
        bslot, bjj = (0, jj) if jj < 8 else (1, jj - 8)
        wait_rows(rslot)
        drain(bslot, bjj, rslot)
        nxt = jj + 2
        if nxt < 8:
          fire(0, nxt, rslot)
        elif nxt == 8:
          wait_block(1)
          fire(1, 0, rslot)
        elif nxt < 16:
          fire(1, nxt - 8, rslot)

    @pl.when(c == 1)
    def _deg_pipeline():
      load_block(0, 0)
      load_block(1, 1)

      def dstep(p, carry):
        for bslot in range(2):
          wait_block(bslot)
          for jj in range(NB):
            pltpu.sync_copy(ones_v, acc_sh.at[dstb_v.at[bslot, jj]],
                            add=True)
          load_block(2 * p + 2 + bslot, bslot)
        return carry

      lax.fori_loop(0, NBLOCKS // 2 - 1, dstep, 0)
      for bslot in range(2):
        wait_block(bslot)
        for jj in range(NB):
          pltpu.sync_copy(ones_v, acc_sh.at[dstb_v.at[bslot, jj]], add=True)

    plsc.subcore_barrier()

    @pl.when(c == 0)
    def _write_sums():
      for kk in range(8):
        pltpu.sync_copy(acc_sh.at[pl.ds(zbase + kk * SROWS, SROWS)], stage_v)
        pltpu.sync_copy(stage_v,
                        psum_hbm.at[pl.ds(zbase + kk * SROWS, SROWS)])

    @pl.when(c == 1)
    def _write_degs():
      for kk in range(8):
        pltpu.sync_copy(acc_sh.at[pl.ds(zbase + kk * SROWS, SROWS)], stage_v)
        pltpu.sync_copy(stage_v,
                        pdeg_hbm.at[pl.ds(zbase + kk * SROWS, SROWS)])

  return k(x, srcdst, ones_blk, zeros_blk)


BLK_N = 1000


def _tc_body(p_ref, d_ref, x_ref, wl_ref, wr_ref, b_ref, o_ref):
  deg = jnp.maximum(d_ref[...][:, 0:1], 1.0)
  aggr = p_ref[...] / deg
  o_ref[...] = (
      jnp.dot(aggr, wl_ref[...], preferred_element_type=jnp.float32)
      + jnp.dot(x_ref[...], wr_ref[...], preferred_element_type=jnp.float32)
      + b_ref[...]
  )


def kernel(x, edge_index, W_l, b_l, W_r):
  pad = E_PAD - N_EDGES
  src = jnp.concatenate([edge_index[0], jnp.zeros((pad,), jnp.int32)])
  dst = jnp.concatenate(
      [edge_index[1], jnp.full((pad,), N_NODES, jnp.int32)])
  srcdst = jnp.stack([src, dst]).reshape(
      2, NUM_SUBCORES, NBLOCKS, NB, CHUNK)

  ones_blk = jnp.ones((CHUNK, D_IN), jnp.float32)
  zeros_blk = jnp.zeros((SROWS, D_IN), jnp.float32)

  psum, pdeg = _sc_aggregate(x, srcdst, ones_blk, zeros_blk)

  grid = N_NODES // BLK_N
  out = pl.pallas_call(
      _tc_body,
      grid=(grid,),
      in_specs=[
          pl.BlockSpec((BLK_N, D_IN), lambda i: (i, 0)),
          pl.BlockSpec((BLK_N, D_IN), lambda i: (i, 0)),
          pl.BlockSpec((BLK_N, D_IN), lambda i: (i, 0)),
          pl.BlockSpec((D_IN, D_OUT), lambda i: (0, 0)),
          pl.BlockSpec((D_IN, D_OUT), lambda i: (0, 0)),
          pl.BlockSpec((1, D_OUT), lambda i: (0, 0)),
      ],
      out_specs=pl.BlockSpec((BLK_N, D_OUT), lambda i: (i, 0)),
      out_shape=jax.ShapeDtypeStruct((N_NODES, D_OUT), jnp.float32),
  )(psum, pdeg, x, W_l.T, W_r.T, b_l.reshape(1, D_OUT))
  return out


# speedup vs baseline: 4.4375x; 1.0410x over previous
import functools

import jax
import jax.numpy as jnp
from jax import lax
from jax.experimental import pallas as pl
from jax.experimental.pallas import tpu as pltpu
from jax.experimental.pallas import tpu_sc as plsc

N_NODES = 10000
N_EDGES = 320000
D_IN = 128
D_OUT = 128

NUM_CORES = 2
NUM_SUBCORES = 16

CHUNK = 64
NB = 8
CHUNKS_PER_TILE = 160
NBLOCKS = CHUNKS_PER_TILE // NB
EDGES_PER_TILE = CHUNK * CHUNKS_PER_TILE
E_PAD = EDGES_PER_TILE * NUM_SUBCORES * NUM_CORES
N_PAD = 10240
ZROWS = N_PAD // NUM_SUBCORES
SROWS = ZROWS // 8


def _sc_aggregate(x, srcdst, ones_blk, zeros_blk):
  mesh = plsc.VectorSubcoreMesh(core_axis_name="c", subcore_axis_name="s")

  @functools.partial(
      pl.kernel,
      out_type=(
          jax.ShapeDtypeStruct((NUM_CORES, N_PAD, D_IN), jnp.float32),
          jax.ShapeDtypeStruct((NUM_CORES, N_PAD, D_IN), jnp.float32),
      ),
      mesh=mesh,
      scratch_types=[
          pltpu.VMEM((2, NB, CHUNK), jnp.int32),
          pltpu.VMEM((2, NB, CHUNK), jnp.int32),
          pltpu.VMEM((2, CHUNK, D_IN), jnp.float32),
          pltpu.VMEM((CHUNK, D_IN), jnp.float32),
          pltpu.VMEM((SROWS, D_IN), jnp.float32),
          pltpu.VMEM_SHARED((N_PAD, D_IN), jnp.float32),
          pltpu.SemaphoreType.DMA,
          pltpu.SemaphoreType.DMA,
          pltpu.SemaphoreType.DMA,
          pltpu.SemaphoreType.DMA,
      ],
  )
  def k(x_hbm, sd_hbm, ones_hbm, z_hbm, psum_hbm, pdeg_hbm,
        srcb_v, dstb_v, rows_v, ones_v, stage_v, acc_sh,
        rsem0, rsem1, isem0, isem1):
    c = lax.axis_index("c")
    s = lax.axis_index("s")
    rsems = (rsem0, rsem1)
    isems = (isem0, isem1)
    zbase = s * ZROWS

    def zero_acc():
      for kk in range(8):
        pltpu.sync_copy(stage_v, acc_sh.at[pl.ds(zbase + kk * SROWS, SROWS)])

    def write_acc(out_hbm):
      for kk in range(8):
        pltpu.sync_copy(acc_sh.at[pl.ds(zbase + kk * SROWS, SROWS)], stage_v)
        pltpu.sync_copy(stage_v,
                        out_hbm.at[c, pl.ds(zbase + kk * SROWS, SROWS)])

    def load_block(g, slot, with_src):
      if with_src:
        pltpu.async_copy(sd_hbm.at[0, c, s, g], srcb_v.at[slot], isems[slot])
      pltpu.async_copy(sd_hbm.at[1, c, s, g], dstb_v.at[slot], isems[slot])

    def wait_block(slot, with_src):
      if with_src:
        pltpu.make_async_copy(sd_hbm.at[0, c, s, 0], srcb_v.at[slot],
                              isems[slot]).wait()
      pltpu.make_async_copy(sd_hbm.at[1, c, s, 0], dstb_v.at[slot],
                            isems[slot]).wait()

    pltpu.sync_copy(ones_hbm, ones_v)
    pltpu.sync_copy(z_hbm, stage_v)
    zero_acc()
    plsc.subcore_barrier()

    def fire(bslot, jj, rslot):
      pltpu.async_copy(x_hbm.at[srcb_v.at[bslot, jj]], rows_v.at[rslot],
                       rsems[rslot])

    def wait_rows(rslot):
      pltpu.make_async_copy(x_hbm.at[srcb_v.at[0, 0]], rows_v.at[rslot],
                            rsems[rslot]).wait()

    def drain(bslot, jj, rslot):
      pltpu.sync_copy(rows_v.at[rslot], acc_sh.at[dstb_v.at[bslot, jj]],
                      add=True)

    load_block(0, 0, True)
    load_block(1, 1, True)
    wait_block(0, True)
    fire(0, 0, 0)
    fire(0, 1, 1)

    def step(p, carry):
      for jj in range(16):
        rslot = jj & 1
        bslot, bjj = (0, jj) if jj < 8 else (1, jj - 8)
        wait_rows(rslot)
        drain(bslot, bjj, rslot)
        nxt = jj + 2
        if nxt < 8:
          fire(0, nxt, rslot)
        elif nxt == 8:
          wait_block(1, True)
          fire(1, 0, rslot)
        elif nxt == 9:
          load_block(2 * p + 2, 0, True)
          fire(1, 1, rslot)
        elif nxt < 16:
          fire(1, nxt - 8, rslot)
        elif nxt == 16:
          wait_block(0, True)
          fire(0, 0, rslot)
        else:
          load_block(2 * p + 3, 1, True)
          fire(0, 1, rslot)
      return carry

    lax.fori_loop(0, NBLOCKS // 2 - 1, step, 0)

    for jj in range(16):
      rslot = jj & 1
      bslot, bjj = (0, jj) if jj < 8 else (1, jj - 8)
      wait_rows(rslot)
      drain(bslot, bjj, rslot)
      nxt = jj + 2
      if nxt < 8:
        fire(0, nxt, rslot)
      elif nxt == 8:
        wait_block(1, True)
        fire(1, 0, rslot)
      elif nxt < 16:
        fire(1, nxt - 8, rslot)

    plsc.subcore_barrier()
    write_acc(psum_hbm)
    plsc.subcore_barrier()
    pltpu.sync_copy(z_hbm, stage_v)
    zero_acc()
    plsc.subcore_barrier()

    load_block(0, 0, False)
    load_block(1, 1, False)

    def dstep(p, carry):
      for bslot in range(2):
        wait_block(bslot, False)
        for jj in range(NB):
          pltpu.sync_copy(ones_v, acc_sh.at[dstb_v.at[bslot, jj]], add=True)
        load_block(2 * p + 2 + bslot, bslot, False)
      return carry

    lax.fori_loop(0, NBLOCKS // 2 - 1, dstep, 0)
    for bslot in range(2):
      wait_block(bslot, False)
      for jj in range(NB):
        pltpu.sync_copy(ones_v, acc_sh.at[dstb_v.at[bslot, jj]], add=True)

    plsc.subcore_barrier()
    write_acc(pdeg_hbm)

  return k(x, srcdst, ones_blk, zeros_blk)


BLK_N = 1000


def _tc_body(p0_ref, p1_ref, d0_ref, d1_ref, x_ref, wl_ref, wr_ref, b_ref,
             o_ref):
  deg = jnp.maximum((d0_ref[0] + d1_ref[0])[:, 0:1], 1.0)
  aggr = (p0_ref[0] + p1_ref[0]) / deg
  o_ref[...] = (
      jnp.dot(aggr, wl_ref[...], preferred_element_type=jnp.float32)
      + jnp.dot(x_ref[...], wr_ref[...], preferred_element_type=jnp.float32)
      + b_ref[...]
  )


def kernel(x, edge_index, W_l, b_l, W_r):
  pad = E_PAD - N_EDGES
  src = jnp.concatenate([edge_index[0], jnp.zeros((pad,), jnp.int32)])
  dst = jnp.concatenate(
      [edge_index[1], jnp.full((pad,), N_NODES, jnp.int32)])
  srcdst = jnp.stack([src, dst]).reshape(
      2, NUM_CORES, NUM_SUBCORES, NBLOCKS, NB, CHUNK)

  ones_blk = jnp.ones((CHUNK, D_IN), jnp.float32)
  zeros_blk = jnp.zeros((SROWS, D_IN), jnp.float32)

  psum, pdeg = _sc_aggregate(x, srcdst, ones_blk, zeros_blk)

  grid = N_NODES // BLK_N
  out = pl.pallas_call(
      _tc_body,
      grid=(grid,),
      in_specs=[
          pl.BlockSpec((1, BLK_N, D_IN), lambda i: (0, i, 0)),
          pl.BlockSpec((1, BLK_N, D_IN), lambda i: (1, i, 0)),
          pl.BlockSpec((1, BLK_N, D_IN), lambda i: (0, i, 0)),
          pl.BlockSpec((1, BLK_N, D_IN), lambda i: (1, i, 0)),
          pl.BlockSpec((BLK_N, D_IN), lambda i: (i, 0)),
          pl.BlockSpec((D_IN, D_OUT), lambda i: (0, 0)),
          pl.BlockSpec((D_IN, D_OUT), lambda i: (0, 0)),
          pl.BlockSpec((1, D_OUT), lambda i: (0, 0)),
      ],
      out_specs=pl.BlockSpec((BLK_N, D_OUT), lambda i: (i, 0)),
      out_shape=jax.ShapeDtypeStruct((N_NODES, D_OUT), jnp.float32),
  )(psum, psum, pdeg, pdeg, x, W_l.T, W_r.T, b_l.reshape(1, D_OUT))
  return out
